# baseline (speedup 1.0000x reference)
import functools

import jax
import jax.numpy as jnp
from jax import lax
from jax.experimental import pallas as pl
from jax.experimental.pallas import tpu as pltpu
from jax.experimental.pallas import tpu_sc as plsc

_NUM_CORES = 2
_NUM_SUBCORES = 16
_NUM_WORKERS = _NUM_CORES * _NUM_SUBCORES
_NBUF = 4


@functools.partial(jax.jit, static_argnums=(1, 2, 3))
def _pos_broadcast(pos_embedding, B, N, D):
    rows_per_w = N // _NUM_WORKERS
    ch = rows_per_w
    while ch * D * 4 * _NBUF > 448 * 1024:
        ch //= 2
    n_chunks = rows_per_w // ch

    mesh = plsc.VectorSubcoreMesh(core_axis_name="c", subcore_axis_name="s")

    @functools.partial(
        pl.kernel,
        out_type=jax.ShapeDtypeStruct((B * N, D), jnp.float32),
        mesh=mesh,
        scratch_types=[
            [pltpu.VMEM((ch, D), jnp.float32) for _ in range(_NBUF)],
            pltpu.SemaphoreType.DMA,
            pltpu.SemaphoreType.DMA,
        ],
    )
    def k(table_hbm, out_hbm, bufs, gsem, ssem):
        wid = lax.axis_index("s") * _NUM_CORES + lax.axis_index("c")
        base = wid * rows_per_w

        gathers = [None] * n_chunks
        scatters = {}
        for i in range(min(_NBUF, n_chunks)):
            gathers[i] = pltpu.async_copy(
                table_hbm.at[pl.ds(base + i * ch, ch)], bufs[i], gsem)
        for i in range(n_chunks):
            gathers[i].wait()
            row0 = base + i * ch
            scatters[i] = [
                pltpu.async_copy(
                    bufs[i % _NBUF], out_hbm.at[pl.ds(b * N + row0, ch)],
                    ssem)
                for b in range(B)
            ]
            j = i + _NBUF
            if j < n_chunks:
                for c in scatters.pop(i):
                    c.wait()
                gathers[j] = pltpu.async_copy(
                    table_hbm.at[pl.ds(base + j * ch, ch)],
                    bufs[j % _NBUF], gsem)
        for cs in scatters.values():
            for c in cs:
                c.wait()

    return k(pos_embedding)


def kernel(x, pos_embedding):
    B, N, D = x.shape
    out = _pos_broadcast(pos_embedding, B, N, D)
    return out.reshape(B, N, D)


# speedup vs baseline: 1.0384x; 1.0384x over previous
import functools

import jax
import jax.numpy as jnp
from jax import lax
from jax.experimental import pallas as pl
from jax.experimental.pallas import tpu as pltpu
from jax.experimental.pallas import tpu_sc as plsc

_NUM_CORES = 2
_NUM_SUBCORES = 16
_NUM_WORKERS = _NUM_CORES * _NUM_SUBCORES


@functools.partial(jax.jit, static_argnums=(1, 2, 3))
def _pos_broadcast(pos_embedding, B, N, D):
    rows_per_w = N // _NUM_WORKERS
    ch = rows_per_w
    while ch * D * 4 * 2 > 384 * 1024:
        ch //= 2
    n_chunks = rows_per_w // ch

    mesh = plsc.VectorSubcoreMesh(core_axis_name="c", subcore_axis_name="s")

    @functools.partial(
        pl.kernel,
        out_type=jax.ShapeDtypeStruct((B * N, D), jnp.float32),
        mesh=mesh,
        scratch_types=[
            pltpu.VMEM((ch, D), jnp.float32),
            pltpu.VMEM((ch, D), jnp.float32),
            pltpu.SemaphoreType.DMA,
            pltpu.SemaphoreType.DMA,
        ],
    )
    def k(table_hbm, out_hbm, buf0, buf1, gsem, ssem):
        wid = lax.axis_index("s") * _NUM_CORES + lax.axis_index("c")
        base = wid * rows_per_w
        bufs = (buf0, buf1)

        gathers = [None] * n_chunks
        scatters = {}
        gathers[0] = pltpu.async_copy(
            table_hbm.at[pl.ds(base, ch)], bufs[0], gsem)
        for i in range(n_chunks):
            if i + 1 < n_chunks:
                for c in scatters.pop(i - 1, ()):
                    c.wait()
                gathers[i + 1] = pltpu.async_copy(
                    table_hbm.at[pl.ds(base + (i + 1) * ch, ch)],
                    bufs[(i + 1) % 2], gsem)
            gathers[i].wait()
            row0 = base + i * ch
            scatters[i] = [
                pltpu.async_copy(
                    bufs[i % 2], out_hbm.at[pl.ds(b * N + row0, ch)], ssem)
                for b in range(B)
            ]
        for cs in scatters.values():
            for c in cs:
                c.wait()

    return k(pos_embedding)


def kernel(x, pos_embedding):
    B, N, D = x.shape
    out = _pos_broadcast(pos_embedding, B, N, D)
    return out.reshape(B, N, D)


# speedup vs baseline: 1.0482x; 1.0094x over previous
import functools

import jax
import jax.numpy as jnp
from jax import lax
from jax.experimental import pallas as pl
from jax.experimental.pallas import tpu as pltpu
from jax.experimental.pallas import tpu_sc as plsc

_NUM_CORES = 2
_NUM_SUBCORES = 16
_NUM_WORKERS = _NUM_CORES * _NUM_SUBCORES


@functools.partial(jax.jit, static_argnums=(1, 2, 3))
def _pos_broadcast(pos_embedding, B, N, D):
    rows_per_w = N // _NUM_WORKERS
    ch = rows_per_w
    while ch * D * 4 * 3 > 448 * 1024:
        ch //= 2
    n_chunks = rows_per_w // ch

    mesh = plsc.VectorSubcoreMesh(core_axis_name="c", subcore_axis_name="s")

    @functools.partial(
        pl.kernel,
        out_type=jax.ShapeDtypeStruct((B * N, D), jnp.float32),
        mesh=mesh,
        scratch_types=[
            pltpu.VMEM((ch, D), jnp.float32),
            pltpu.VMEM((ch, D), jnp.float32),
            pltpu.VMEM((ch, D), jnp.float32),
            pltpu.SemaphoreType.DMA,
            pltpu.SemaphoreType.DMA,
        ],
    )
    def k(table_hbm, out_hbm, buf0, buf1, buf2, gsem, ssem):
        wid = lax.axis_index("s") * _NUM_CORES + lax.axis_index("c")
        base = wid * rows_per_w
        bufs = (buf0, buf1, buf2)
        nbuf = len(bufs)

        gathers = [None] * n_chunks
        scatters = {}
        for i in range(min(nbuf, n_chunks)):
            gathers[i] = pltpu.async_copy(
                table_hbm.at[pl.ds(base + i * ch, ch)], bufs[i], gsem)
        for i in range(n_chunks):
            gathers[i].wait()
            row0 = base + i * ch
            scatters[i] = [
                pltpu.async_copy(
                    bufs[i % nbuf], out_hbm.at[pl.ds(b * N + row0, ch)],
                    ssem)
                for b in range(B)
            ]
            j = i + nbuf
            if j < n_chunks:
                for c in scatters.pop(i):
                    c.wait()
                gathers[j] = pltpu.async_copy(
                    table_hbm.at[pl.ds(base + j * ch, ch)],
                    bufs[j % nbuf], gsem)
        for cs in scatters.values():
            for c in cs:
                c.wait()

    return k(pos_embedding)


def kernel(x, pos_embedding):
    B, N, D = x.shape
    out = _pos_broadcast(pos_embedding, B, N, D)
    return out.reshape(B, N, D)
